# baseline (speedup 1.0000x reference)
import functools

import jax
import jax.numpy as jnp
from jax import lax
from jax.experimental import pallas as pl
from jax.experimental.pallas import tpu as pltpu
from jax.experimental.pallas import tpu_sc as plsc

N = 10000
D = 128
E = 320000
NC = 2
NS = 16
NW = NC * NS
E_PER_W = E // NW
C = 80
NCHUNK = E_PER_W // C
NPAD = 10240
ROWS_PER_TILE = NPAD // NS
RB = 16

HIST_PER_TILE = NPAD // NS


def _deg_body(src_hbm, dst_hbm, ones_hbm, zdeg_hbm, out_hbm,
              degsh_s, degsh_d, idx_v, ones_v, dbounce):
    cid = lax.axis_index("c")
    sid = lax.axis_index("s")
    w = cid * NS + sid

    pltpu.sync_copy(ones_hbm, ones_v)
    pltpu.sync_copy(zdeg_hbm, dbounce)
    pltpu.sync_copy(dbounce, degsh_s.at[pl.ds(sid * HIST_PER_TILE, HIST_PER_TILE)])
    pltpu.sync_copy(dbounce, degsh_d.at[pl.ds(sid * HIST_PER_TILE, HIST_PER_TILE)])
    plsc.subcore_barrier()

    base = w * E_PER_W

    def body(g, carry):
        eb = base + g * C
        pltpu.sync_copy(src_hbm.at[pl.ds(eb, C)], idx_v)
        pltpu.sync_copy(ones_v, degsh_s.at[idx_v], add=True)
        pltpu.sync_copy(dst_hbm.at[pl.ds(eb, C)], idx_v)
        pltpu.sync_copy(ones_v, degsh_d.at[idx_v], add=True)
        return carry

    lax.fori_loop(0, NCHUNK, body, 0)
    plsc.subcore_barrier()

    r0 = sid * HIST_PER_TILE
    pltpu.sync_copy(degsh_s.at[pl.ds(r0, HIST_PER_TILE)], dbounce)
    pltpu.sync_copy(dbounce, out_hbm.at[cid, 0, pl.ds(r0, HIST_PER_TILE)])
    pltpu.sync_copy(degsh_d.at[pl.ds(r0, HIST_PER_TILE)], dbounce)
    pltpu.sync_copy(dbounce, out_hbm.at[cid, 1, pl.ds(r0, HIST_PER_TILE)])


CP = 80
KBUF = 3
NROUND0 = 29
NROUND1 = 55
F0 = NROUND0 * KBUF
F1 = NROUND1 * KBUF
E_PAD = (F0 + F1) * NS * CP
OUTB = ROWS_PER_TILE // RB


def _agg_body(hn_hbm, src_hbm, dst_hbm, out_hbm,
              aggsh, sidx, didx, rows, bounce, semi, semg, sems, semo):
    cid = lax.axis_index("c")
    sid = lax.axis_index("s")
    w = cid * NS + sid

    zeros16 = jnp.zeros((16,), jnp.float32)

    def zstore(i, carry):
        bounce[0][i // 8, pl.ds((i % 8) * 16, 16)] = zeros16
        return carry

    lax.fori_loop(0, RB * 8, zstore, 0)
    zd = []
    for j in range(OUTB):
        zd.append(pltpu.async_copy(
            bounce[0], aggsh.at[pl.ds(sid * ROWS_PER_TILE + j * RB, RB)],
            semo.at[0]))
    for d in zd:
        d.wait()
    plsc.subcore_barrier()

    base = jnp.where(cid == 0, sid * F0, NS * F0 + sid * F1) * CP
    nrounds = jnp.where(cid == 0, NROUND0, NROUND1)

    for b in range(KBUF):
        eb = base + b * CP
        pltpu.async_copy(src_hbm.at[pl.ds(eb, CP)], sidx[b], semi.at[b])
        pltpu.async_copy(dst_hbm.at[pl.ds(eb, CP)], didx[b], semi.at[b])

    def round_body(g, carry):
        eb0 = base + g * KBUF * CP
        for b in range(KBUF):
            pltpu.make_async_copy(
                src_hbm.at[pl.ds(eb0 + b * CP, CP)], sidx[b], semi.at[b]).wait()
            pltpu.make_async_copy(
                dst_hbm.at[pl.ds(eb0 + b * CP, CP)], didx[b], semi.at[b]).wait()
        gdescs = [
            pltpu.async_copy(hn_hbm.at[sidx[b]], rows[b], semg.at[b])
            for b in range(KBUF)
        ]
        sdescs = []
        for b in range(KBUF):
            gdescs[b].wait()
            sdescs.append(
                pltpu.async_copy(rows[b], aggsh.at[didx[b]], sems.at[b], add=True))
        for b in range(KBUF):
            sdescs[b].wait()

        @pl.when(g < nrounds - 1)
        def _prefetch():
            ebn = base + (g + 1) * KBUF * CP
            for b in range(KBUF):
                pltpu.async_copy(src_hbm.at[pl.ds(ebn + b * CP, CP)], sidx[b],
                                 semi.at[b])
                pltpu.async_copy(dst_hbm.at[pl.ds(ebn + b * CP, CP)], didx[b],
                                 semi.at[b])
        return carry

    lax.fori_loop(0, nrounds, round_body, 0)
    plsc.subcore_barrier()

    odescs = [None, None]
    for j in range(OUTB):
        b2 = j % 2
        if odescs[b2] is not None:
            odescs[b2].wait()
        r0 = sid * ROWS_PER_TILE + j * RB
        pltpu.sync_copy(aggsh.at[pl.ds(r0, RB)], bounce[b2])
        odescs[b2] = pltpu.async_copy(bounce[b2], out_hbm.at[cid, pl.ds(r0, RB)],
                                      semo.at[b2])
    for d in odescs:
        if d is not None:
            d.wait()


@functools.cache
def _sc_kernels():
    mesh = plsc.VectorSubcoreMesh(
        core_axis_name="c", subcore_axis_name="s", num_cores=NC, num_subcores=NS
    )
    params = pltpu.CompilerParams(use_tc_tiling_on_sc=False)
    deg = functools.partial(
        pl.kernel,
        out_type=jax.ShapeDtypeStruct((NC, 2, NPAD), jnp.float32),
        mesh=mesh,
        compiler_params=params,
        scratch_types=[
            pltpu.VMEM_SHARED((NPAD,), jnp.float32),
            pltpu.VMEM_SHARED((NPAD,), jnp.float32),
            pltpu.VMEM((C,), jnp.int32),
            pltpu.VMEM((C,), jnp.float32),
            pltpu.VMEM((HIST_PER_TILE,), jnp.float32),
        ],
    )(_deg_body)
    agg = functools.partial(
        pl.kernel,
        out_type=jax.ShapeDtypeStruct((NC, NPAD, D), jnp.float32),
        mesh=mesh,
        compiler_params=params,
        scratch_types=[
            pltpu.VMEM_SHARED((NPAD, D), jnp.float32),
            [pltpu.VMEM((CP,), jnp.int32) for _ in range(KBUF)],
            [pltpu.VMEM((CP,), jnp.int32) for _ in range(KBUF)],
            [pltpu.VMEM((CP, D), jnp.float32) for _ in range(KBUF)],
            [pltpu.VMEM((RB, D), jnp.float32) for _ in range(2)],
            pltpu.SemaphoreType.DMA((KBUF,)),
            pltpu.SemaphoreType.DMA((KBUF,)),
            pltpu.SemaphoreType.DMA((KBUF,)),
            pltpu.SemaphoreType.DMA((2,)),
        ],
    )(_agg_body)
    return deg, agg


R = 400


def _norm_body(degs_ref, x_ref, ns_ref, nd_ref, hn_ref):
    degs = jnp.sum(degs_ref[...], axis=0)
    ns = lax.rsqrt(jnp.maximum(degs[0], 1.0))
    nd = lax.rsqrt(jnp.maximum(degs[1], 1.0))
    ns_ref[...] = ns
    nd_ref[...] = nd
    hn_ref[...] = x_ref[...] * ns


_norm_call = pl.pallas_call(
    _norm_body,
    grid=(N // R,),
    in_specs=[
        pl.BlockSpec((NC, 2, R, 1), lambda i: (0, 0, i, 0)),
        pl.BlockSpec((R, D), lambda i: (i, 0)),
    ],
    out_specs=[
        pl.BlockSpec((R, 1), lambda i: (i, 0)),
        pl.BlockSpec((R, 1), lambda i: (i, 0)),
        pl.BlockSpec((R, D), lambda i: (i, 0)),
    ],
    out_shape=[
        jax.ShapeDtypeStruct((N, 1), jnp.float32),
        jax.ShapeDtypeStruct((N, 1), jnp.float32),
        jax.ShapeDtypeStruct((N, D), jnp.float32),
    ],
)


def _mm_body(h_ref, agg_ref, nd_ref, ns_ref, w_ref, b_ref,
             hout_ref, hn_ref, *, elu):
    agg = (agg_ref[0] + agg_ref[1]) * nd_ref[...]
    z = jnp.dot(agg, w_ref[...], preferred_element_type=jnp.float32) + b_ref[...]
    if elu:
        z = jnp.where(z > 0, z, jnp.exp(jnp.minimum(z, 0.0)) - 1.0)
    hnew = h_ref[...] + z
    hout_ref[...] = hnew
    hn_ref[...] = hnew * ns_ref[...]


def _make_mm(elu):
    return pl.pallas_call(
        functools.partial(_mm_body, elu=elu),
        grid=(N // R,),
        in_specs=[
            pl.BlockSpec((R, D), lambda i: (i, 0)),
            pl.BlockSpec((NC, R, D), lambda i: (0, i, 0)),
            pl.BlockSpec((R, 1), lambda i: (i, 0)),
            pl.BlockSpec((R, 1), lambda i: (i, 0)),
            pl.BlockSpec((D, D), lambda i: (0, 0)),
            pl.BlockSpec((1, D), lambda i: (0, 0)),
        ],
        out_specs=[
            pl.BlockSpec((R, D), lambda i: (i, 0)),
            pl.BlockSpec((R, D), lambda i: (i, 0)),
        ],
        out_shape=[
            jax.ShapeDtypeStruct((N, D), jnp.float32),
            jax.ShapeDtypeStruct((N, D), jnp.float32),
        ],
    )


_mm_elu = _make_mm(True)
_mm_lin = _make_mm(False)


def kernel(x, edge_index, W0, b0, W1, b1, W2, b2):
    src = edge_index[0].astype(jnp.int32)
    dst = edge_index[1].astype(jnp.int32)
    ones1d = jnp.ones((C,), jnp.float32)
    zdeg = jnp.zeros((HIST_PER_TILE,), jnp.float32)
    _deg_kernel, _agg_kernel = _sc_kernels()

    degs = _deg_kernel(src, dst, ones1d, zdeg)
    degs = degs[:, :, :N].reshape(NC, 2, N, 1)
    norm_s, norm_d, hn = _norm_call(degs, x)

    npad_e = E_PAD - E
    src_p = jnp.concatenate([src, jnp.zeros((npad_e,), jnp.int32)])
    dst_p = jnp.concatenate([dst, jnp.full((npad_e,), N + 16, jnp.int32)])

    h = x
    for (W, b, mm) in ((W0, b0, _mm_elu), (W1, b1, _mm_elu), (W2, b2, _mm_lin)):
        agg2 = _agg_kernel(hn, src_p, dst_p)
        h, hn = mm(h, agg2, norm_d, norm_s, W, b.reshape(1, D))
    return h


# speedup vs baseline: 1.2563x; 1.2563x over previous
import functools

import jax
import jax.numpy as jnp
from jax import lax
from jax.experimental import pallas as pl
from jax.experimental.pallas import tpu as pltpu
from jax.experimental.pallas import tpu_sc as plsc

N = 10000
D = 128
E = 320000
NC = 2
NS = 16
NW = NC * NS
E_PER_W = E // NW
C = 80
NCHUNK = E_PER_W // C
NPAD = 10240
ROWS_PER_TILE = NPAD // NS
RB = 16

HIST_PER_TILE = NPAD // NS


def _deg_body(src_hbm, dst_hbm, ones_hbm, zdeg_hbm, out_hbm,
              degsh_s, degsh_d, idx_v, ones_v, dbounce):
    cid = lax.axis_index("c")
    sid = lax.axis_index("s")
    w = cid * NS + sid

    pltpu.sync_copy(ones_hbm, ones_v)
    pltpu.sync_copy(zdeg_hbm, dbounce)
    pltpu.sync_copy(dbounce, degsh_s.at[pl.ds(sid * HIST_PER_TILE, HIST_PER_TILE)])
    pltpu.sync_copy(dbounce, degsh_d.at[pl.ds(sid * HIST_PER_TILE, HIST_PER_TILE)])
    plsc.subcore_barrier()

    base = w * E_PER_W

    def body(g, carry):
        eb = base + g * C
        pltpu.sync_copy(src_hbm.at[pl.ds(eb, C)], idx_v)
        pltpu.sync_copy(ones_v, degsh_s.at[idx_v], add=True)
        pltpu.sync_copy(dst_hbm.at[pl.ds(eb, C)], idx_v)
        pltpu.sync_copy(ones_v, degsh_d.at[idx_v], add=True)
        return carry

    lax.fori_loop(0, NCHUNK, body, 0)
    plsc.subcore_barrier()

    r0 = sid * HIST_PER_TILE
    pltpu.sync_copy(degsh_s.at[pl.ds(r0, HIST_PER_TILE)], dbounce)
    pltpu.sync_copy(dbounce, out_hbm.at[cid, 0, pl.ds(r0, HIST_PER_TILE)])
    pltpu.sync_copy(degsh_d.at[pl.ds(r0, HIST_PER_TILE)], dbounce)
    pltpu.sync_copy(dbounce, out_hbm.at[cid, 1, pl.ds(r0, HIST_PER_TILE)])


CP = 80
KBUF = 3
NROUND0 = 55
NROUND1 = 29
F0 = NROUND0 * KBUF
F1 = NROUND1 * KBUF
E_PAD = (F0 + F1) * NS * CP
OUTB = ROWS_PER_TILE // RB


def _agg_body(hn_hbm, src_hbm, dst_hbm, out_hbm,
              aggsh, sidx, didx, rows, bounce, semi, semg, sems, semo):
    cid = lax.axis_index("c")
    sid = lax.axis_index("s")
    w = cid * NS + sid

    zeros16 = jnp.zeros((16,), jnp.float32)

    def zstore(i, carry):
        bounce[0][i // 8, pl.ds((i % 8) * 16, 16)] = zeros16
        return carry

    lax.fori_loop(0, RB * 8, zstore, 0)
    zd = []
    for j in range(OUTB):
        zd.append(pltpu.async_copy(
            bounce[0], aggsh.at[pl.ds(sid * ROWS_PER_TILE + j * RB, RB)],
            semo.at[0]))
    for d in zd:
        d.wait()
    plsc.subcore_barrier()

    base = jnp.where(cid == 0, sid * F0, NS * F0 + sid * F1) * CP
    nrounds = jnp.where(cid == 0, NROUND0, NROUND1)

    for b in range(KBUF):
        eb = base + b * CP
        pltpu.async_copy(src_hbm.at[pl.ds(eb, CP)], sidx[b], semi.at[b])
        pltpu.async_copy(dst_hbm.at[pl.ds(eb, CP)], didx[b], semi.at[b])

    def round_body(g, carry):
        eb0 = base + g * KBUF * CP
        for b in range(KBUF):
            pltpu.make_async_copy(
                src_hbm.at[pl.ds(eb0 + b * CP, CP)], sidx[b], semi.at[b]).wait()
            pltpu.make_async_copy(
                dst_hbm.at[pl.ds(eb0 + b * CP, CP)], didx[b], semi.at[b]).wait()
        gdescs = [
            pltpu.async_copy(hn_hbm.at[sidx[b]], rows[b], semg.at[b])
            for b in range(KBUF)
        ]
        sdescs = []
        for b in range(KBUF):
            gdescs[b].wait()
            sdescs.append(
                pltpu.async_copy(rows[b], aggsh.at[didx[b]], sems.at[b], add=True))
        for b in range(KBUF):
            sdescs[b].wait()

        @pl.when(g < nrounds - 1)
        def _prefetch():
            ebn = base + (g + 1) * KBUF * CP
            for b in range(KBUF):
                pltpu.async_copy(src_hbm.at[pl.ds(ebn + b * CP, CP)], sidx[b],
                                 semi.at[b])
                pltpu.async_copy(dst_hbm.at[pl.ds(ebn + b * CP, CP)], didx[b],
                                 semi.at[b])
        return carry

    lax.fori_loop(0, nrounds, round_body, 0)
    plsc.subcore_barrier()

    odescs = [None, None]
    for j in range(OUTB):
        b2 = j % 2
        if odescs[b2] is not None:
            odescs[b2].wait()
        r0 = sid * ROWS_PER_TILE + j * RB
        pltpu.sync_copy(aggsh.at[pl.ds(r0, RB)], bounce[b2])
        odescs[b2] = pltpu.async_copy(bounce[b2], out_hbm.at[cid, pl.ds(r0, RB)],
                                      semo.at[b2])
    for d in odescs:
        if d is not None:
            d.wait()


@functools.cache
def _sc_kernels():
    mesh = plsc.VectorSubcoreMesh(
        core_axis_name="c", subcore_axis_name="s", num_cores=NC, num_subcores=NS
    )
    params = pltpu.CompilerParams(use_tc_tiling_on_sc=False)
    deg = functools.partial(
        pl.kernel,
        out_type=jax.ShapeDtypeStruct((NC, 2, NPAD), jnp.float32),
        mesh=mesh,
        compiler_params=params,
        scratch_types=[
            pltpu.VMEM_SHARED((NPAD,), jnp.float32),
            pltpu.VMEM_SHARED((NPAD,), jnp.float32),
            pltpu.VMEM((C,), jnp.int32),
            pltpu.VMEM((C,), jnp.float32),
            pltpu.VMEM((HIST_PER_TILE,), jnp.float32),
        ],
    )(_deg_body)
    agg = functools.partial(
        pl.kernel,
        out_type=jax.ShapeDtypeStruct((NC, NPAD, D), jnp.float32),
        mesh=mesh,
        compiler_params=params,
        scratch_types=[
            pltpu.VMEM_SHARED((NPAD, D), jnp.float32),
            [pltpu.VMEM((CP,), jnp.int32) for _ in range(KBUF)],
            [pltpu.VMEM((CP,), jnp.int32) for _ in range(KBUF)],
            [pltpu.VMEM((CP, D), jnp.float32) for _ in range(KBUF)],
            [pltpu.VMEM((RB, D), jnp.float32) for _ in range(2)],
            pltpu.SemaphoreType.DMA((KBUF,)),
            pltpu.SemaphoreType.DMA((KBUF,)),
            pltpu.SemaphoreType.DMA((KBUF,)),
            pltpu.SemaphoreType.DMA((2,)),
        ],
    )(_agg_body)
    return deg, agg


R = 400


def _norm_body(degs_ref, x_ref, ns_ref, nd_ref, hn_ref):
    degs = jnp.sum(degs_ref[...], axis=0)
    ns = lax.rsqrt(jnp.maximum(degs[0], 1.0))
    nd = lax.rsqrt(jnp.maximum(degs[1], 1.0))
    ns_ref[...] = ns
    nd_ref[...] = nd
    hn_ref[...] = x_ref[...] * ns


_norm_call = pl.pallas_call(
    _norm_body,
    grid=(N // R,),
    in_specs=[
        pl.BlockSpec((NC, 2, R, 1), lambda i: (0, 0, i, 0)),
        pl.BlockSpec((R, D), lambda i: (i, 0)),
    ],
    out_specs=[
        pl.BlockSpec((R, 1), lambda i: (i, 0)),
        pl.BlockSpec((R, 1), lambda i: (i, 0)),
        pl.BlockSpec((R, D), lambda i: (i, 0)),
    ],
    out_shape=[
        jax.ShapeDtypeStruct((N, 1), jnp.float32),
        jax.ShapeDtypeStruct((N, 1), jnp.float32),
        jax.ShapeDtypeStruct((N, D), jnp.float32),
    ],
)


def _mm_body(h_ref, agg_ref, nd_ref, ns_ref, w_ref, b_ref,
             hout_ref, hn_ref, *, elu):
    agg = (agg_ref[0] + agg_ref[1]) * nd_ref[...]
    z = jnp.dot(agg, w_ref[...], preferred_element_type=jnp.float32) + b_ref[...]
    if elu:
        z = jnp.where(z > 0, z, jnp.exp(jnp.minimum(z, 0.0)) - 1.0)
    hnew = h_ref[...] + z
    hout_ref[...] = hnew
    hn_ref[...] = hnew * ns_ref[...]


def _make_mm(elu):
    return pl.pallas_call(
        functools.partial(_mm_body, elu=elu),
        grid=(N // R,),
        in_specs=[
            pl.BlockSpec((R, D), lambda i: (i, 0)),
            pl.BlockSpec((NC, R, D), lambda i: (0, i, 0)),
            pl.BlockSpec((R, 1), lambda i: (i, 0)),
            pl.BlockSpec((R, 1), lambda i: (i, 0)),
            pl.BlockSpec((D, D), lambda i: (0, 0)),
            pl.BlockSpec((1, D), lambda i: (0, 0)),
        ],
        out_specs=[
            pl.BlockSpec((R, D), lambda i: (i, 0)),
            pl.BlockSpec((R, D), lambda i: (i, 0)),
        ],
        out_shape=[
            jax.ShapeDtypeStruct((N, D), jnp.float32),
            jax.ShapeDtypeStruct((N, D), jnp.float32),
        ],
    )


_mm_elu = _make_mm(True)
_mm_lin = _make_mm(False)


def kernel(x, edge_index, W0, b0, W1, b1, W2, b2):
    src = edge_index[0].astype(jnp.int32)
    dst = edge_index[1].astype(jnp.int32)
    ones1d = jnp.ones((C,), jnp.float32)
    zdeg = jnp.zeros((HIST_PER_TILE,), jnp.float32)
    _deg_kernel, _agg_kernel = _sc_kernels()

    degs = _deg_kernel(src, dst, ones1d, zdeg)
    degs = degs[:, :, :N].reshape(NC, 2, N, 1)
    norm_s, norm_d, hn = _norm_call(degs, x)

    npad_e = E_PAD - E
    src_p = jnp.concatenate([src, jnp.zeros((npad_e,), jnp.int32)])
    dst_p = jnp.concatenate([dst, jnp.full((npad_e,), N + 16, jnp.int32)])

    h = x
    for (W, b, mm) in ((W0, b0, _mm_elu), (W1, b1, _mm_elu), (W2, b2, _mm_lin)):
        agg2 = _agg_kernel(hn, src_p, dst_p)
        h, hn = mm(h, agg2, norm_d, norm_s, W, b.reshape(1, D))
    return h


# speedup vs baseline: 1.2950x; 1.0308x over previous
import functools

import jax
import jax.numpy as jnp
from jax import lax
from jax.experimental import pallas as pl
from jax.experimental.pallas import tpu as pltpu
from jax.experimental.pallas import tpu_sc as plsc

N = 10000
D = 128
E = 320000
NC = 2
NS = 16
NW = NC * NS
E_PER_W = E // NW
C = 80
NCHUNK = E_PER_W // C
NPAD = 10240
ROWS_PER_TILE = NPAD // NS
RB = 16

HIST_PER_TILE = NPAD // NS


def _deg_body(src_hbm, dst_hbm, ones_hbm, zdeg_hbm, out_hbm,
              degsh_s, degsh_d, idx_v, ones_v, dbounce):
    cid = lax.axis_index("c")
    sid = lax.axis_index("s")
    w = cid * NS + sid

    pltpu.sync_copy(ones_hbm, ones_v)
    pltpu.sync_copy(zdeg_hbm, dbounce)
    pltpu.sync_copy(dbounce, degsh_s.at[pl.ds(sid * HIST_PER_TILE, HIST_PER_TILE)])
    pltpu.sync_copy(dbounce, degsh_d.at[pl.ds(sid * HIST_PER_TILE, HIST_PER_TILE)])
    plsc.subcore_barrier()

    base = w * E_PER_W

    def body(g, carry):
        eb = base + g * C
        pltpu.sync_copy(src_hbm.at[pl.ds(eb, C)], idx_v)
        pltpu.sync_copy(ones_v, degsh_s.at[idx_v], add=True)
        pltpu.sync_copy(dst_hbm.at[pl.ds(eb, C)], idx_v)
        pltpu.sync_copy(ones_v, degsh_d.at[idx_v], add=True)
        return carry

    lax.fori_loop(0, NCHUNK, body, 0)
    plsc.subcore_barrier()

    r0 = sid * HIST_PER_TILE
    pltpu.sync_copy(degsh_s.at[pl.ds(r0, HIST_PER_TILE)], dbounce)
    pltpu.sync_copy(dbounce, out_hbm.at[cid, 0, pl.ds(r0, HIST_PER_TILE)])
    pltpu.sync_copy(degsh_d.at[pl.ds(r0, HIST_PER_TILE)], dbounce)
    pltpu.sync_copy(dbounce, out_hbm.at[cid, 1, pl.ds(r0, HIST_PER_TILE)])


CP = 80
KBUF = 3
NROUND0 = 59
NROUND1 = 25
F0 = NROUND0 * KBUF
F1 = NROUND1 * KBUF
E_PAD = (F0 + F1) * NS * CP
OUTB = ROWS_PER_TILE // RB


def _agg_body(hn_hbm, src_hbm, dst_hbm, out_hbm,
              aggsh, sidx, didx, rows, bounce, semi, semg, sems, semo):
    cid = lax.axis_index("c")
    sid = lax.axis_index("s")
    w = cid * NS + sid

    zeros16 = jnp.zeros((16,), jnp.float32)

    def zstore(i, carry):
        bounce[0][i // 8, pl.ds((i % 8) * 16, 16)] = zeros16
        return carry

    lax.fori_loop(0, RB * 8, zstore, 0)
    zd = []
    for j in range(OUTB):
        zd.append(pltpu.async_copy(
            bounce[0], aggsh.at[pl.ds(sid * ROWS_PER_TILE + j * RB, RB)],
            semo.at[0]))
    for d in zd:
        d.wait()
    plsc.subcore_barrier()

    base = jnp.where(cid == 0, sid * F0, NS * F0 + sid * F1) * CP
    nrounds = jnp.where(cid == 0, NROUND0, NROUND1)

    for b in range(KBUF):
        eb = base + b * CP
        pltpu.async_copy(src_hbm.at[pl.ds(eb, CP)], sidx[b], semi.at[b])
        pltpu.async_copy(dst_hbm.at[pl.ds(eb, CP)], didx[b], semi.at[b])

    def round_body(g, carry):
        eb0 = base + g * KBUF * CP
        for b in range(KBUF):
            pltpu.make_async_copy(
                src_hbm.at[pl.ds(eb0 + b * CP, CP)], sidx[b], semi.at[b]).wait()
            pltpu.make_async_copy(
                dst_hbm.at[pl.ds(eb0 + b * CP, CP)], didx[b], semi.at[b]).wait()
        gdescs = [
            pltpu.async_copy(hn_hbm.at[sidx[b]], rows[b], semg.at[b])
            for b in range(KBUF)
        ]
        sdescs = []
        for b in range(KBUF):
            gdescs[b].wait()
            sdescs.append(
                pltpu.async_copy(rows[b], aggsh.at[didx[b]], sems.at[b], add=True))
        for b in range(KBUF):
            sdescs[b].wait()

        @pl.when(g < nrounds - 1)
        def _prefetch():
            ebn = base + (g + 1) * KBUF * CP
            for b in range(KBUF):
                pltpu.async_copy(src_hbm.at[pl.ds(ebn + b * CP, CP)], sidx[b],
                                 semi.at[b])
                pltpu.async_copy(dst_hbm.at[pl.ds(ebn + b * CP, CP)], didx[b],
                                 semi.at[b])
        return carry

    lax.fori_loop(0, nrounds, round_body, 0)
    plsc.subcore_barrier()

    odescs = [None, None]
    for j in range(OUTB):
        b2 = j % 2
        if odescs[b2] is not None:
            odescs[b2].wait()
        r0 = sid * ROWS_PER_TILE + j * RB
        pltpu.sync_copy(aggsh.at[pl.ds(r0, RB)], bounce[b2])
        odescs[b2] = pltpu.async_copy(bounce[b2], out_hbm.at[cid, pl.ds(r0, RB)],
                                      semo.at[b2])
    for d in odescs:
        if d is not None:
            d.wait()


@functools.cache
def _sc_kernels():
    mesh = plsc.VectorSubcoreMesh(
        core_axis_name="c", subcore_axis_name="s", num_cores=NC, num_subcores=NS
    )
    params = pltpu.CompilerParams(use_tc_tiling_on_sc=False)
    deg = functools.partial(
        pl.kernel,
        out_type=jax.ShapeDtypeStruct((NC, 2, NPAD), jnp.float32),
        mesh=mesh,
        compiler_params=params,
        scratch_types=[
            pltpu.VMEM_SHARED((NPAD,), jnp.float32),
            pltpu.VMEM_SHARED((NPAD,), jnp.float32),
            pltpu.VMEM((C,), jnp.int32),
            pltpu.VMEM((C,), jnp.float32),
            pltpu.VMEM((HIST_PER_TILE,), jnp.float32),
        ],
    )(_deg_body)
    agg = functools.partial(
        pl.kernel,
        out_type=jax.ShapeDtypeStruct((NC, NPAD, D), jnp.float32),
        mesh=mesh,
        compiler_params=params,
        scratch_types=[
            pltpu.VMEM_SHARED((NPAD, D), jnp.float32),
            [pltpu.VMEM((CP,), jnp.int32) for _ in range(KBUF)],
            [pltpu.VMEM((CP,), jnp.int32) for _ in range(KBUF)],
            [pltpu.VMEM((CP, D), jnp.float32) for _ in range(KBUF)],
            [pltpu.VMEM((RB, D), jnp.float32) for _ in range(2)],
            pltpu.SemaphoreType.DMA((KBUF,)),
            pltpu.SemaphoreType.DMA((KBUF,)),
            pltpu.SemaphoreType.DMA((KBUF,)),
            pltpu.SemaphoreType.DMA((2,)),
        ],
    )(_agg_body)
    return deg, agg


R = 400


def _norm_body(degs_ref, x_ref, ns_ref, nd_ref, hn_ref):
    degs = jnp.sum(degs_ref[...], axis=0)
    ns = lax.rsqrt(jnp.maximum(degs[0], 1.0))
    nd = lax.rsqrt(jnp.maximum(degs[1], 1.0))
    ns_ref[...] = ns
    nd_ref[...] = nd
    hn_ref[...] = x_ref[...] * ns


_norm_call = pl.pallas_call(
    _norm_body,
    grid=(N // R,),
    in_specs=[
        pl.BlockSpec((NC, 2, R, 1), lambda i: (0, 0, i, 0)),
        pl.BlockSpec((R, D), lambda i: (i, 0)),
    ],
    out_specs=[
        pl.BlockSpec((R, 1), lambda i: (i, 0)),
        pl.BlockSpec((R, 1), lambda i: (i, 0)),
        pl.BlockSpec((R, D), lambda i: (i, 0)),
    ],
    out_shape=[
        jax.ShapeDtypeStruct((N, 1), jnp.float32),
        jax.ShapeDtypeStruct((N, 1), jnp.float32),
        jax.ShapeDtypeStruct((N, D), jnp.float32),
    ],
)


def _mm_body(h_ref, agg_ref, nd_ref, ns_ref, w_ref, b_ref,
             hout_ref, hn_ref, *, elu):
    agg = (agg_ref[0] + agg_ref[1]) * nd_ref[...]
    z = jnp.dot(agg, w_ref[...], preferred_element_type=jnp.float32) + b_ref[...]
    if elu:
        z = jnp.where(z > 0, z, jnp.exp(jnp.minimum(z, 0.0)) - 1.0)
    hnew = h_ref[...] + z
    hout_ref[...] = hnew
    hn_ref[...] = hnew * ns_ref[...]


def _make_mm(elu):
    return pl.pallas_call(
        functools.partial(_mm_body, elu=elu),
        grid=(N // R,),
        in_specs=[
            pl.BlockSpec((R, D), lambda i: (i, 0)),
            pl.BlockSpec((NC, R, D), lambda i: (0, i, 0)),
            pl.BlockSpec((R, 1), lambda i: (i, 0)),
            pl.BlockSpec((R, 1), lambda i: (i, 0)),
            pl.BlockSpec((D, D), lambda i: (0, 0)),
            pl.BlockSpec((1, D), lambda i: (0, 0)),
        ],
        out_specs=[
            pl.BlockSpec((R, D), lambda i: (i, 0)),
            pl.BlockSpec((R, D), lambda i: (i, 0)),
        ],
        out_shape=[
            jax.ShapeDtypeStruct((N, D), jnp.float32),
            jax.ShapeDtypeStruct((N, D), jnp.float32),
        ],
    )


_mm_elu = _make_mm(True)
_mm_lin = _make_mm(False)


def kernel(x, edge_index, W0, b0, W1, b1, W2, b2):
    src = edge_index[0].astype(jnp.int32)
    dst = edge_index[1].astype(jnp.int32)
    ones1d = jnp.ones((C,), jnp.float32)
    zdeg = jnp.zeros((HIST_PER_TILE,), jnp.float32)
    _deg_kernel, _agg_kernel = _sc_kernels()

    degs = _deg_kernel(src, dst, ones1d, zdeg)
    degs = degs[:, :, :N].reshape(NC, 2, N, 1)
    norm_s, norm_d, hn = _norm_call(degs, x)

    npad_e = E_PAD - E
    src_p = jnp.concatenate([src, jnp.zeros((npad_e,), jnp.int32)])
    dst_p = jnp.concatenate([dst, jnp.full((npad_e,), N + 16, jnp.int32)])

    h = x
    for (W, b, mm) in ((W0, b0, _mm_elu), (W1, b1, _mm_elu), (W2, b2, _mm_lin)):
        agg2 = _agg_kernel(hn, src_p, dst_p)
        h, hn = mm(h, agg2, norm_d, norm_s, W, b.reshape(1, D))
    return h


# speedup vs baseline: 1.4629x; 1.1297x over previous
import functools

import jax
import jax.numpy as jnp
from jax import lax
from jax.experimental import pallas as pl
from jax.experimental.pallas import tpu as pltpu
from jax.experimental.pallas import tpu_sc as plsc

N = 10000
D = 128
E = 320000
NC = 2
NS = 16
NW = NC * NS
E_PER_W = E // NW
C = 80
NCHUNK = E_PER_W // C
NPAD = 10240
ROWS_PER_TILE = NPAD // NS
RB = 16

HIST_PER_TILE = NPAD // NS
CD = 1000
NDR = E_PER_W // CD


def _deg_body(src_hbm, dst_hbm, ones_hbm, zdeg_hbm, out_hbm,
              degsh_s, degsh_d, isl, idl, ones_v, dbounce,
              semi2, sema, semb):
    cid = lax.axis_index("c")
    sid = lax.axis_index("s")
    w = cid * NS + sid

    pltpu.sync_copy(ones_hbm, ones_v)
    pltpu.sync_copy(zdeg_hbm, dbounce)
    pltpu.sync_copy(dbounce, degsh_s.at[pl.ds(sid * HIST_PER_TILE, HIST_PER_TILE)])
    pltpu.sync_copy(dbounce, degsh_d.at[pl.ds(sid * HIST_PER_TILE, HIST_PER_TILE)])
    plsc.subcore_barrier()

    base = w * E_PER_W

    pltpu.async_copy(src_hbm.at[pl.ds(base, CD)], isl[0], semi2.at[0])
    pltpu.async_copy(dst_hbm.at[pl.ds(base, CD)], idl[0], semi2.at[0])

    def body(g, carry):
        k = lax.rem(g, 2)
        eb = base + g * CD
        for kk in range(2):
            @pl.when(k == kk)
            def _round(kk=kk):
                ko = 1 - kk
                pltpu.make_async_copy(
                    src_hbm.at[pl.ds(eb, CD)], isl[kk], semi2.at[kk]).wait()
                pltpu.make_async_copy(
                    dst_hbm.at[pl.ds(eb, CD)], idl[kk], semi2.at[kk]).wait()
                @pl.when(g > 0)
                def _drain():
                    pltpu.make_async_copy(
                        ones_v, degsh_s.at[isl[ko]], sema.at[ko]).wait()
                    pltpu.make_async_copy(
                        ones_v, degsh_d.at[idl[ko]], semb.at[ko]).wait()

                @pl.when(g < NDR - 1)
                def _prefetch():
                    ebn = eb + CD
                    pltpu.async_copy(src_hbm.at[pl.ds(ebn, CD)], isl[ko],
                                     semi2.at[ko])
                    pltpu.async_copy(dst_hbm.at[pl.ds(ebn, CD)], idl[ko],
                                     semi2.at[ko])
                pltpu.async_copy(ones_v, degsh_s.at[isl[kk]], sema.at[kk],
                                 add=True)
                pltpu.async_copy(ones_v, degsh_d.at[idl[kk]], semb.at[kk],
                                 add=True)
        return carry

    lax.fori_loop(0, NDR, body, 0)
    kl = (NDR - 1) % 2
    pltpu.make_async_copy(ones_v, degsh_s.at[isl[kl]], sema.at[kl]).wait()
    pltpu.make_async_copy(ones_v, degsh_d.at[idl[kl]], semb.at[kl]).wait()
    plsc.subcore_barrier()

    r0 = sid * HIST_PER_TILE
    pltpu.sync_copy(degsh_s.at[pl.ds(r0, HIST_PER_TILE)], dbounce)
    pltpu.sync_copy(dbounce, out_hbm.at[cid, 0, pl.ds(r0, HIST_PER_TILE)])
    pltpu.sync_copy(degsh_d.at[pl.ds(r0, HIST_PER_TILE)], dbounce)
    pltpu.sync_copy(dbounce, out_hbm.at[cid, 1, pl.ds(r0, HIST_PER_TILE)])


CP = 80
KBUF = 3
NROUND0 = 59
NROUND1 = 25
F0 = NROUND0 * KBUF
F1 = NROUND1 * KBUF
E_PAD = (F0 + F1) * NS * CP
OUTB = ROWS_PER_TILE // RB


def _agg_body(hn_hbm, src_hbm, dst_hbm, out_hbm,
              aggsh, sidx, didx, rows, bounce, semi, semg, sems, semo):
    cid = lax.axis_index("c")
    sid = lax.axis_index("s")
    w = cid * NS + sid

    zeros16 = jnp.zeros((16,), jnp.float32)

    def zstore(i, carry):
        bounce[0][i // 8, pl.ds((i % 8) * 16, 16)] = zeros16
        return carry

    lax.fori_loop(0, RB * 8, zstore, 0)
    zd = []
    for j in range(OUTB):
        zd.append(pltpu.async_copy(
            bounce[0], aggsh.at[pl.ds(sid * ROWS_PER_TILE + j * RB, RB)],
            semo.at[0]))
    for d in zd:
        d.wait()
    plsc.subcore_barrier()

    base = jnp.where(cid == 0, sid * F0, NS * F0 + sid * F1) * CP
    nrounds = jnp.where(cid == 0, NROUND0, NROUND1)

    for b in range(KBUF):
        eb = base + b * CP
        pltpu.async_copy(src_hbm.at[pl.ds(eb, CP)], sidx[b], semi.at[b])
        pltpu.async_copy(dst_hbm.at[pl.ds(eb, CP)], didx[b], semi.at[b])

    def round_body(g, carry):
        eb0 = base + g * KBUF * CP
        for b in range(KBUF):
            pltpu.make_async_copy(
                src_hbm.at[pl.ds(eb0 + b * CP, CP)], sidx[b], semi.at[b]).wait()
            pltpu.make_async_copy(
                dst_hbm.at[pl.ds(eb0 + b * CP, CP)], didx[b], semi.at[b]).wait()
        gdescs = [
            pltpu.async_copy(hn_hbm.at[sidx[b]], rows[b], semg.at[b])
            for b in range(KBUF)
        ]
        sdescs = []
        for b in range(KBUF):
            gdescs[b].wait()
            sdescs.append(
                pltpu.async_copy(rows[b], aggsh.at[didx[b]], sems.at[b], add=True))
        for b in range(KBUF):
            sdescs[b].wait()

        @pl.when(g < nrounds - 1)
        def _prefetch():
            ebn = base + (g + 1) * KBUF * CP
            for b in range(KBUF):
                pltpu.async_copy(src_hbm.at[pl.ds(ebn + b * CP, CP)], sidx[b],
                                 semi.at[b])
                pltpu.async_copy(dst_hbm.at[pl.ds(ebn + b * CP, CP)], didx[b],
                                 semi.at[b])
        return carry

    lax.fori_loop(0, nrounds, round_body, 0)
    plsc.subcore_barrier()

    odescs = [None, None]
    for j in range(OUTB):
        b2 = j % 2
        if odescs[b2] is not None:
            odescs[b2].wait()
        r0 = sid * ROWS_PER_TILE + j * RB
        pltpu.sync_copy(aggsh.at[pl.ds(r0, RB)], bounce[b2])
        odescs[b2] = pltpu.async_copy(bounce[b2], out_hbm.at[cid, pl.ds(r0, RB)],
                                      semo.at[b2])
    for d in odescs:
        if d is not None:
            d.wait()


@functools.cache
def _sc_kernels():
    mesh = plsc.VectorSubcoreMesh(
        core_axis_name="c", subcore_axis_name="s", num_cores=NC, num_subcores=NS
    )
    params = pltpu.CompilerParams(use_tc_tiling_on_sc=False)
    deg = functools.partial(
        pl.kernel,
        out_type=jax.ShapeDtypeStruct((NC, 2, NPAD), jnp.float32),
        mesh=mesh,
        compiler_params=params,
        scratch_types=[
            pltpu.VMEM_SHARED((NPAD,), jnp.float32),
            pltpu.VMEM_SHARED((NPAD,), jnp.float32),
            [pltpu.VMEM((CD,), jnp.int32) for _ in range(2)],
            [pltpu.VMEM((CD,), jnp.int32) for _ in range(2)],
            pltpu.VMEM((CD,), jnp.float32),
            pltpu.VMEM((HIST_PER_TILE,), jnp.float32),
            pltpu.SemaphoreType.DMA((2,)),
            pltpu.SemaphoreType.DMA((2,)),
            pltpu.SemaphoreType.DMA((2,)),
        ],
    )(_deg_body)
    agg = functools.partial(
        pl.kernel,
        out_type=jax.ShapeDtypeStruct((NC, NPAD, D), jnp.float32),
        mesh=mesh,
        compiler_params=params,
        scratch_types=[
            pltpu.VMEM_SHARED((NPAD, D), jnp.float32),
            [pltpu.VMEM((CP,), jnp.int32) for _ in range(KBUF)],
            [pltpu.VMEM((CP,), jnp.int32) for _ in range(KBUF)],
            [pltpu.VMEM((CP, D), jnp.float32) for _ in range(KBUF)],
            [pltpu.VMEM((RB, D), jnp.float32) for _ in range(2)],
            pltpu.SemaphoreType.DMA((KBUF,)),
            pltpu.SemaphoreType.DMA((KBUF,)),
            pltpu.SemaphoreType.DMA((KBUF,)),
            pltpu.SemaphoreType.DMA((2,)),
        ],
    )(_agg_body)
    return deg, agg


R = 400


def _norm_body(degs_ref, x_ref, ns_ref, nd_ref, hn_ref):
    degs = jnp.sum(degs_ref[...], axis=0)
    ns = lax.rsqrt(jnp.maximum(degs[0], 1.0))
    nd = lax.rsqrt(jnp.maximum(degs[1], 1.0))
    ns_ref[...] = ns
    nd_ref[...] = nd
    hn_ref[...] = x_ref[...] * ns


_norm_call = pl.pallas_call(
    _norm_body,
    grid=(N // R,),
    in_specs=[
        pl.BlockSpec((NC, 2, R, 1), lambda i: (0, 0, i, 0)),
        pl.BlockSpec((R, D), lambda i: (i, 0)),
    ],
    out_specs=[
        pl.BlockSpec((R, 1), lambda i: (i, 0)),
        pl.BlockSpec((R, 1), lambda i: (i, 0)),
        pl.BlockSpec((R, D), lambda i: (i, 0)),
    ],
    out_shape=[
        jax.ShapeDtypeStruct((N, 1), jnp.float32),
        jax.ShapeDtypeStruct((N, 1), jnp.float32),
        jax.ShapeDtypeStruct((N, D), jnp.float32),
    ],
)


def _mm_body(h_ref, agg_ref, nd_ref, ns_ref, w_ref, b_ref,
             hout_ref, hn_ref, *, elu):
    agg = (agg_ref[0] + agg_ref[1]) * nd_ref[...]
    z = jnp.dot(agg, w_ref[...], preferred_element_type=jnp.float32) + b_ref[...]
    if elu:
        z = jnp.where(z > 0, z, jnp.exp(jnp.minimum(z, 0.0)) - 1.0)
    hnew = h_ref[...] + z
    hout_ref[...] = hnew
    hn_ref[...] = hnew * ns_ref[...]


def _make_mm(elu):
    return pl.pallas_call(
        functools.partial(_mm_body, elu=elu),
        grid=(N // R,),
        in_specs=[
            pl.BlockSpec((R, D), lambda i: (i, 0)),
            pl.BlockSpec((NC, R, D), lambda i: (0, i, 0)),
            pl.BlockSpec((R, 1), lambda i: (i, 0)),
            pl.BlockSpec((R, 1), lambda i: (i, 0)),
            pl.BlockSpec((D, D), lambda i: (0, 0)),
            pl.BlockSpec((1, D), lambda i: (0, 0)),
        ],
        out_specs=[
            pl.BlockSpec((R, D), lambda i: (i, 0)),
            pl.BlockSpec((R, D), lambda i: (i, 0)),
        ],
        out_shape=[
            jax.ShapeDtypeStruct((N, D), jnp.float32),
            jax.ShapeDtypeStruct((N, D), jnp.float32),
        ],
    )


_mm_elu = _make_mm(True)
_mm_lin = _make_mm(False)


def kernel(x, edge_index, W0, b0, W1, b1, W2, b2):
    src = edge_index[0].astype(jnp.int32)
    dst = edge_index[1].astype(jnp.int32)
    ones1d = jnp.ones((CD,), jnp.float32)
    zdeg = jnp.zeros((HIST_PER_TILE,), jnp.float32)
    _deg_kernel, _agg_kernel = _sc_kernels()

    degs = _deg_kernel(src, dst, ones1d, zdeg)
    degs = degs[:, :, :N].reshape(NC, 2, N, 1)
    norm_s, norm_d, hn = _norm_call(degs, x)

    npad_e = E_PAD - E
    src_p = jnp.concatenate([src, jnp.zeros((npad_e,), jnp.int32)])
    dst_p = jnp.concatenate([dst, jnp.full((npad_e,), N + 16, jnp.int32)])

    h = x
    for (W, b, mm) in ((W0, b0, _mm_elu), (W1, b1, _mm_elu), (W2, b2, _mm_lin)):
        agg2 = _agg_kernel(hn, src_p, dst_p)
        h, hn = mm(h, agg2, norm_d, norm_s, W, b.reshape(1, D))
    return h


# speedup vs baseline: 1.4857x; 1.0156x over previous
import functools

import jax
import jax.numpy as jnp
from jax import lax
from jax.experimental import pallas as pl
from jax.experimental.pallas import tpu as pltpu
from jax.experimental.pallas import tpu_sc as plsc

N = 10000
D = 128
E = 320000
NC = 2
NS = 16
NW = NC * NS
E_PER_W = E // NW
C = 80
NCHUNK = E_PER_W // C
NPAD = 10240
ROWS_PER_TILE = NPAD // NS
RB = 16

HIST_PER_TILE = NPAD // NS
CD = 1000
NDR = E_PER_W // CD


def _deg_body(src_hbm, dst_hbm, ones_hbm, zdeg_hbm, out_hbm,
              degsh_s, degsh_d, isl, idl, ones_v, dbounce,
              semi2, sema, semb):
    cid = lax.axis_index("c")
    sid = lax.axis_index("s")
    w = cid * NS + sid

    pltpu.sync_copy(ones_hbm, ones_v)
    pltpu.sync_copy(zdeg_hbm, dbounce)
    pltpu.sync_copy(dbounce, degsh_s.at[pl.ds(sid * HIST_PER_TILE, HIST_PER_TILE)])
    pltpu.sync_copy(dbounce, degsh_d.at[pl.ds(sid * HIST_PER_TILE, HIST_PER_TILE)])
    plsc.subcore_barrier()

    base = w * E_PER_W

    pltpu.async_copy(src_hbm.at[pl.ds(base, CD)], isl[0], semi2.at[0])
    pltpu.async_copy(dst_hbm.at[pl.ds(base, CD)], idl[0], semi2.at[0])

    def body(g, carry):
        k = lax.rem(g, 2)
        eb = base + g * CD
        for kk in range(2):
            @pl.when(k == kk)
            def _round(kk=kk):
                ko = 1 - kk
                pltpu.make_async_copy(
                    src_hbm.at[pl.ds(eb, CD)], isl[kk], semi2.at[kk]).wait()
                pltpu.make_async_copy(
                    dst_hbm.at[pl.ds(eb, CD)], idl[kk], semi2.at[kk]).wait()
                @pl.when(g > 0)
                def _drain():
                    pltpu.make_async_copy(
                        ones_v, degsh_s.at[isl[ko]], sema.at[ko]).wait()
                    pltpu.make_async_copy(
                        ones_v, degsh_d.at[idl[ko]], semb.at[ko]).wait()

                @pl.when(g < NDR - 1)
                def _prefetch():
                    ebn = eb + CD
                    pltpu.async_copy(src_hbm.at[pl.ds(ebn, CD)], isl[ko],
                                     semi2.at[ko])
                    pltpu.async_copy(dst_hbm.at[pl.ds(ebn, CD)], idl[ko],
                                     semi2.at[ko])
                pltpu.async_copy(ones_v, degsh_s.at[isl[kk]], sema.at[kk],
                                 add=True)
                pltpu.async_copy(ones_v, degsh_d.at[idl[kk]], semb.at[kk],
                                 add=True)
        return carry

    lax.fori_loop(0, NDR, body, 0)
    kl = (NDR - 1) % 2
    pltpu.make_async_copy(ones_v, degsh_s.at[isl[kl]], sema.at[kl]).wait()
    pltpu.make_async_copy(ones_v, degsh_d.at[idl[kl]], semb.at[kl]).wait()
    plsc.subcore_barrier()

    r0 = sid * HIST_PER_TILE
    pltpu.sync_copy(degsh_s.at[pl.ds(r0, HIST_PER_TILE)], dbounce)
    pltpu.sync_copy(dbounce, out_hbm.at[cid, 0, pl.ds(r0, HIST_PER_TILE)])
    pltpu.sync_copy(degsh_d.at[pl.ds(r0, HIST_PER_TILE)], dbounce)
    pltpu.sync_copy(dbounce, out_hbm.at[cid, 1, pl.ds(r0, HIST_PER_TILE)])


CP = 80
KBUF = 3
NROUND0 = 61
NROUND1 = 23
F0 = NROUND0 * KBUF
F1 = NROUND1 * KBUF
E_PAD = (F0 + F1) * NS * CP
OUTB = ROWS_PER_TILE // RB


def _agg_body(hn_hbm, src_hbm, dst_hbm, out_hbm,
              aggsh, sidx, didx, rows, bounce, semi, semg, sems, semo):
    cid = lax.axis_index("c")
    sid = lax.axis_index("s")
    w = cid * NS + sid

    zeros16 = jnp.zeros((16,), jnp.float32)

    def zstore(i, carry):
        bounce[0][i // 8, pl.ds((i % 8) * 16, 16)] = zeros16
        return carry

    lax.fori_loop(0, RB * 8, zstore, 0)
    zd = []
    for j in range(OUTB):
        zd.append(pltpu.async_copy(
            bounce[0], aggsh.at[pl.ds(sid * ROWS_PER_TILE + j * RB, RB)],
            semo.at[0]))
    for d in zd:
        d.wait()
    plsc.subcore_barrier()

    base = jnp.where(cid == 0, sid * F0, NS * F0 + sid * F1) * CP
    nrounds = jnp.where(cid == 0, NROUND0, NROUND1)

    for b in range(KBUF):
        eb = base + b * CP
        pltpu.async_copy(src_hbm.at[pl.ds(eb, CP)], sidx[b], semi.at[b])
        pltpu.async_copy(dst_hbm.at[pl.ds(eb, CP)], didx[b], semi.at[b])

    def round_body(g, carry):
        eb0 = base + g * KBUF * CP
        for b in range(KBUF):
            pltpu.make_async_copy(
                src_hbm.at[pl.ds(eb0 + b * CP, CP)], sidx[b], semi.at[b]).wait()
            pltpu.make_async_copy(
                dst_hbm.at[pl.ds(eb0 + b * CP, CP)], didx[b], semi.at[b]).wait()
        gdescs = [
            pltpu.async_copy(hn_hbm.at[sidx[b]], rows[b], semg.at[b])
            for b in range(KBUF)
        ]
        sdescs = []
        for b in range(KBUF):
            gdescs[b].wait()
            sdescs.append(
                pltpu.async_copy(rows[b], aggsh.at[didx[b]], sems.at[b], add=True))
        for b in range(KBUF):
            sdescs[b].wait()

        @pl.when(g < nrounds - 1)
        def _prefetch():
            ebn = base + (g + 1) * KBUF * CP
            for b in range(KBUF):
                pltpu.async_copy(src_hbm.at[pl.ds(ebn + b * CP, CP)], sidx[b],
                                 semi.at[b])
                pltpu.async_copy(dst_hbm.at[pl.ds(ebn + b * CP, CP)], didx[b],
                                 semi.at[b])
        return carry

    lax.fori_loop(0, nrounds, round_body, 0)
    plsc.subcore_barrier()

    odescs = [None, None]
    for j in range(OUTB):
        b2 = j % 2
        if odescs[b2] is not None:
            odescs[b2].wait()
        r0 = sid * ROWS_PER_TILE + j * RB
        pltpu.sync_copy(aggsh.at[pl.ds(r0, RB)], bounce[b2])
        odescs[b2] = pltpu.async_copy(bounce[b2], out_hbm.at[cid, pl.ds(r0, RB)],
                                      semo.at[b2])
    for d in odescs:
        if d is not None:
            d.wait()


@functools.cache
def _sc_kernels():
    mesh = plsc.VectorSubcoreMesh(
        core_axis_name="c", subcore_axis_name="s", num_cores=NC, num_subcores=NS
    )
    params = pltpu.CompilerParams(use_tc_tiling_on_sc=False)
    deg = functools.partial(
        pl.kernel,
        out_type=jax.ShapeDtypeStruct((NC, 2, NPAD), jnp.float32),
        mesh=mesh,
        compiler_params=params,
        scratch_types=[
            pltpu.VMEM_SHARED((NPAD,), jnp.float32),
            pltpu.VMEM_SHARED((NPAD,), jnp.float32),
            [pltpu.VMEM((CD,), jnp.int32) for _ in range(2)],
            [pltpu.VMEM((CD,), jnp.int32) for _ in range(2)],
            pltpu.VMEM((CD,), jnp.float32),
            pltpu.VMEM((HIST_PER_TILE,), jnp.float32),
            pltpu.SemaphoreType.DMA((2,)),
            pltpu.SemaphoreType.DMA((2,)),
            pltpu.SemaphoreType.DMA((2,)),
        ],
    )(_deg_body)
    agg = functools.partial(
        pl.kernel,
        out_type=jax.ShapeDtypeStruct((NC, NPAD, D), jnp.float32),
        mesh=mesh,
        compiler_params=params,
        scratch_types=[
            pltpu.VMEM_SHARED((NPAD, D), jnp.float32),
            [pltpu.VMEM((CP,), jnp.int32) for _ in range(KBUF)],
            [pltpu.VMEM((CP,), jnp.int32) for _ in range(KBUF)],
            [pltpu.VMEM((CP, D), jnp.float32) for _ in range(KBUF)],
            [pltpu.VMEM((RB, D), jnp.float32) for _ in range(2)],
            pltpu.SemaphoreType.DMA((KBUF,)),
            pltpu.SemaphoreType.DMA((KBUF,)),
            pltpu.SemaphoreType.DMA((KBUF,)),
            pltpu.SemaphoreType.DMA((2,)),
        ],
    )(_agg_body)
    return deg, agg


R = 400


def _norm_body(degs_ref, x_ref, ns_ref, nd_ref, hn_ref):
    degs = jnp.sum(degs_ref[...], axis=0)
    ns = lax.rsqrt(jnp.maximum(degs[0], 1.0))
    nd = lax.rsqrt(jnp.maximum(degs[1], 1.0))
    ns_ref[...] = ns
    nd_ref[...] = nd
    hn_ref[...] = x_ref[...] * ns


_norm_call = pl.pallas_call(
    _norm_body,
    grid=(N // R,),
    in_specs=[
        pl.BlockSpec((NC, 2, R, 1), lambda i: (0, 0, i, 0)),
        pl.BlockSpec((R, D), lambda i: (i, 0)),
    ],
    out_specs=[
        pl.BlockSpec((R, 1), lambda i: (i, 0)),
        pl.BlockSpec((R, 1), lambda i: (i, 0)),
        pl.BlockSpec((R, D), lambda i: (i, 0)),
    ],
    out_shape=[
        jax.ShapeDtypeStruct((N, 1), jnp.float32),
        jax.ShapeDtypeStruct((N, 1), jnp.float32),
        jax.ShapeDtypeStruct((N, D), jnp.float32),
    ],
)


def _mm_body(h_ref, agg_ref, nd_ref, ns_ref, w_ref, b_ref,
             hout_ref, hn_ref, *, elu):
    agg = (agg_ref[0] + agg_ref[1]) * nd_ref[...]
    z = jnp.dot(agg, w_ref[...], preferred_element_type=jnp.float32) + b_ref[...]
    if elu:
        z = jnp.where(z > 0, z, jnp.exp(jnp.minimum(z, 0.0)) - 1.0)
    hnew = h_ref[...] + z
    hout_ref[...] = hnew
    hn_ref[...] = hnew * ns_ref[...]


def _make_mm(elu):
    return pl.pallas_call(
        functools.partial(_mm_body, elu=elu),
        grid=(N // R,),
        in_specs=[
            pl.BlockSpec((R, D), lambda i: (i, 0)),
            pl.BlockSpec((NC, R, D), lambda i: (0, i, 0)),
            pl.BlockSpec((R, 1), lambda i: (i, 0)),
            pl.BlockSpec((R, 1), lambda i: (i, 0)),
            pl.BlockSpec((D, D), lambda i: (0, 0)),
            pl.BlockSpec((1, D), lambda i: (0, 0)),
        ],
        out_specs=[
            pl.BlockSpec((R, D), lambda i: (i, 0)),
            pl.BlockSpec((R, D), lambda i: (i, 0)),
        ],
        out_shape=[
            jax.ShapeDtypeStruct((N, D), jnp.float32),
            jax.ShapeDtypeStruct((N, D), jnp.float32),
        ],
    )


_mm_elu = _make_mm(True)
_mm_lin = _make_mm(False)


def kernel(x, edge_index, W0, b0, W1, b1, W2, b2):
    src = edge_index[0].astype(jnp.int32)
    dst = edge_index[1].astype(jnp.int32)
    ones1d = jnp.ones((CD,), jnp.float32)
    zdeg = jnp.zeros((HIST_PER_TILE,), jnp.float32)
    _deg_kernel, _agg_kernel = _sc_kernels()

    degs = _deg_kernel(src, dst, ones1d, zdeg)
    degs = degs[:, :, :N].reshape(NC, 2, N, 1)
    norm_s, norm_d, hn = _norm_call(degs, x)

    npad_e = E_PAD - E
    src_p = jnp.concatenate([src, jnp.zeros((npad_e,), jnp.int32)])
    dst_p = jnp.concatenate([dst, jnp.full((npad_e,), N + 16, jnp.int32)])

    h = x
    for (W, b, mm) in ((W0, b0, _mm_elu), (W1, b1, _mm_elu), (W2, b2, _mm_lin)):
        agg2 = _agg_kernel(hn, src_p, dst_p)
        h, hn = mm(h, agg2, norm_d, norm_s, W, b.reshape(1, D))
    return h
